# baseline (speedup 1.0000x reference)
import functools
import math

import jax
import jax.numpy as jnp
from jax import lax
from jax.experimental import pallas as pl
from jax.experimental.pallas import tpu as pltpu
from jax.experimental.pallas import tpu_sc as plsc

D_MODEL = 64
SCALE = math.sqrt(D_MODEL)
LANES = 16
NBUF = 4


def _row_segments(m: int):
    segs = []
    off = 0
    while m - off > 128:
        segs.append((off, 104))
        off += 104
    segs.append((off, m - off))
    return tuple(segs)


@functools.lru_cache(maxsize=None)
def _build(nw: int, nc: int, n: int, m: int):
    mesh = plsc.VectorSubcoreMesh(core_axis_name="c", subcore_axis_name="s")
    D = D_MODEL
    rows_w = n // nw
    segs = _row_segments(m)
    nbuf = next(b for b in (NBUF, 2, 1) if rows_w % b == 0)

    @functools.partial(
        pl.kernel,
        out_type=jax.ShapeDtypeStruct((n * m, D), jnp.float32),
        mesh=mesh,
        scratch_types=[
            pltpu.VMEM((rows_w, m), jnp.int32),
            pltpu.VMEM((nbuf, m, D), jnp.float32),
            pltpu.VMEM((nbuf, m, D), jnp.float32),
        ]
        + [pltpu.SemaphoreType.DMA] * (2 * nbuf),
        compiler_params=pltpu.CompilerParams(use_tc_tiling_on_sc=False),
    )
    def k(idx_hbm, table_hbm, out_hbm, idx_v, gbuf, obuf, *sems):
        gsem, ssem = sems[:nbuf], sems[nbuf:]
        wid = lax.axis_index("s") * nc + lax.axis_index("c")
        r0 = wid * rows_w
        pltpu.sync_copy(idx_hbm.at[pl.ds(r0, rows_w)], idx_v)

        def gather_row(r, b):
            for off, sz in segs:
                pltpu.async_copy(
                    table_hbm.at[idx_v.at[r, pl.ds(off, sz)]],
                    gbuf.at[b, pl.ds(off, sz)],
                    gsem[b],
                )

        def wait_gather_row(r, b):
            for off, sz in segs:
                pltpu.make_async_copy(
                    table_hbm.at[idx_v.at[r, pl.ds(off, sz)]],
                    gbuf.at[b, pl.ds(off, sz)],
                    gsem[b],
                ).wait()

        for b in range(nbuf):
            gather_row(b, b)

        @pl.loop(0, rows_w, step=nbuf)
        def _outer(g0):
            for b in range(nbuf):
                r = g0 + b
                wait_gather_row(r, b)

                @pl.when(g0 > 0)
                def _():
                    pltpu.make_async_copy(
                        obuf.at[b],
                        out_hbm.at[pl.ds((r0 + r) * m, m)],
                        ssem[b],
                    ).wait()

                @pl.loop(0, m, unroll=4)
                def _row(i):
                    for j in range(D // LANES):
                        sl = pl.ds(j * LANES, LANES)
                        obuf[b, i, sl] = gbuf[b, i, sl] * SCALE

                pltpu.async_copy(
                    obuf.at[b], out_hbm.at[pl.ds((r0 + r) * m, m)], ssem[b]
                )

                @pl.when(g0 + nbuf < rows_w)
                def _():
                    gather_row(r + nbuf, b)

        for b in range(nbuf):
            r = rows_w - nbuf + b
            pltpu.make_async_copy(
                obuf.at[b], out_hbm.at[pl.ds((r0 + r) * m, m)], ssem[b]
            ).wait()

    return k


def kernel(x, table):
    info = plsc.get_sparse_core_info()
    nc, ns = info.num_cores, info.num_subcores
    nw = nc * ns
    orig_shape = x.shape
    xi = x.astype(jnp.int32)
    if xi.ndim == 2 and xi.shape[0] % nw == 0 and xi.shape[1] % 8 == 0:
        n, m = xi.shape
    else:
        b = xi.size
        xf = xi.reshape(-1)
        pad = (-b) % (nw * 8)
        if pad:
            xf = jnp.concatenate([xf, jnp.zeros((pad,), jnp.int32)])
        n, m = nw, (b + pad) // nw
        xi = xf.reshape(n, m)
    out = _build(nw, nc, n, m)(xi, table)
    out = out.reshape(n, m, D_MODEL)
    if (n, m) != orig_shape and orig_shape != (n, m):
        out = out.reshape(-1, D_MODEL)[: x.size]
    return out.reshape(*orig_shape, D_MODEL)


# speedup vs baseline: 1.1855x; 1.1855x over previous
import functools
import math

import jax
import jax.numpy as jnp
from jax import lax
from jax.experimental import pallas as pl
from jax.experimental.pallas import tpu as pltpu
from jax.experimental.pallas import tpu_sc as plsc

D_MODEL = 64
SCALE = math.sqrt(D_MODEL)
LANES = 16
CHUNK = 128
NBUF = 2


@functools.lru_cache(maxsize=None)
def _build(nw: int, nc: int, nids: int, vpad: int):
    mesh = plsc.VectorSubcoreMesh(core_axis_name="c", subcore_axis_name="s")
    D = D_MODEL
    ids_w = nids // nw
    nchunks = ids_w // CHUNK
    nbuf = next(b for b in (NBUF, 2, 1) if nchunks % b == 0)

    @functools.partial(
        pl.kernel,
        out_type=jax.ShapeDtypeStruct((nids, D), jnp.float32),
        mesh=mesh,
        scratch_types=[
            pltpu.VMEM((ids_w,), jnp.int32),
            pltpu.VMEM((nbuf, CHUNK, 128), jnp.float32),
            pltpu.VMEM((nbuf, CHUNK, D), jnp.float32),
        ]
        + [pltpu.SemaphoreType.DMA] * (2 * nbuf),
    )
    def k(idx_hbm, table_hbm, out_hbm, idx_v, gbuf, obuf, *sems):
        gsem, ssem = sems[:nbuf], sems[nbuf:]
        wid = lax.axis_index("s") * nc + lax.axis_index("c")
        base = wid * ids_w
        pltpu.sync_copy(idx_hbm.at[pl.ds(base, ids_w)], idx_v)

        def gather(c, b):
            pltpu.async_copy(
                table_hbm.at[idx_v.at[pl.ds(c * CHUNK, CHUNK)]],
                gbuf.at[b],
                gsem[b],
            )

        for b in range(nbuf):
            gather(b, b)

        @pl.loop(0, nchunks, step=nbuf)
        def _outer(c0):
            for b in range(nbuf):
                c = c0 + b
                pltpu.make_async_copy(
                    table_hbm.at[idx_v.at[pl.ds(c * CHUNK, CHUNK)]],
                    gbuf.at[b],
                    gsem[b],
                ).wait()

                @pl.when(c0 > 0)
                def _():
                    pltpu.make_async_copy(
                        obuf.at[b],
                        out_hbm.at[pl.ds(base + c * CHUNK, CHUNK)],
                        ssem[b],
                    ).wait()

                @pl.loop(0, CHUNK, unroll=4)
                def _row(i):
                    for j in range(D // LANES):
                        sl = pl.ds(j * LANES, LANES)
                        obuf[b, i, sl] = gbuf[b, i, sl] * SCALE

                pltpu.async_copy(
                    obuf.at[b],
                    out_hbm.at[pl.ds(base + c * CHUNK, CHUNK)],
                    ssem[b],
                )

                @pl.when(c0 + nbuf < nchunks)
                def _():
                    gather(c + nbuf, b)

        for b in range(nbuf):
            c = nchunks - nbuf + b
            pltpu.make_async_copy(
                obuf.at[b],
                out_hbm.at[pl.ds(base + c * CHUNK, CHUNK)],
                ssem[b],
            ).wait()

    return k


def kernel(x, table):
    info = plsc.get_sparse_core_info()
    nc, ns = info.num_cores, info.num_subcores
    nw = nc * ns
    orig_shape = x.shape
    b = x.size
    xf = x.reshape(-1).astype(jnp.int32)
    block = nw * CHUNK
    pad = (-b) % block
    if pad:
        xf = jnp.concatenate([xf, jnp.zeros((pad,), jnp.int32)])
    tp = jnp.pad(table, ((0, 0), (0, 128 - D_MODEL)))
    out = _build(nw, nc, b + pad, tp.shape[0])(xf, tp)
    if pad:
        out = out[:b]
    return out.reshape(*orig_shape, D_MODEL)


# speedup vs baseline: 1.3401x; 1.1304x over previous
import functools
import math

import jax
import jax.numpy as jnp
from jax import lax
from jax.experimental import pallas as pl
from jax.experimental.pallas import tpu as pltpu
from jax.experimental.pallas import tpu_sc as plsc

D_MODEL = 64
SCALE = math.sqrt(D_MODEL)
LANES = 16
CHUNK = 80
NBUF = 4


@functools.lru_cache(maxsize=None)
def _build(nw: int, nc: int, nids: int, vpad: int):
    mesh = plsc.VectorSubcoreMesh(core_axis_name="c", subcore_axis_name="s")
    D = D_MODEL
    ids_w = nids // nw
    nchunks = ids_w // CHUNK
    nbuf = next(b for b in (NBUF, 2, 1) if nchunks % b == 0)

    @functools.partial(
        pl.kernel,
        out_type=jax.ShapeDtypeStruct((nids, D), jnp.float32),
        mesh=mesh,
        scratch_types=[
            pltpu.VMEM((ids_w,), jnp.int32),
            pltpu.VMEM((nbuf, CHUNK, 128), jnp.float32),
            pltpu.VMEM((nbuf, CHUNK, D), jnp.float32),
        ]
        + [pltpu.SemaphoreType.DMA] * (2 * nbuf),
    )
    def k(idx_hbm, table_hbm, out_hbm, idx_v, gbuf, obuf, *sems):
        gsem, ssem = sems[:nbuf], sems[nbuf:]
        wid = lax.axis_index("s") * nc + lax.axis_index("c")
        base = wid * ids_w
        pltpu.sync_copy(idx_hbm.at[pl.ds(base, ids_w)], idx_v)

        def gather(c, b):
            pltpu.async_copy(
                table_hbm.at[idx_v.at[pl.ds(c * CHUNK, CHUNK)]],
                gbuf.at[b],
                gsem[b],
            )

        for b in range(nbuf):
            gather(b, b)

        @pl.loop(0, nchunks, step=nbuf)
        def _outer(c0):
            for b in range(nbuf):
                c = c0 + b
                pltpu.make_async_copy(
                    table_hbm.at[idx_v.at[pl.ds(c * CHUNK, CHUNK)]],
                    gbuf.at[b],
                    gsem[b],
                ).wait()

                @pl.when(c0 > 0)
                def _():
                    pltpu.make_async_copy(
                        obuf.at[b],
                        out_hbm.at[pl.ds(base + c * CHUNK, CHUNK)],
                        ssem[b],
                    ).wait()

                @pl.loop(0, CHUNK, unroll=8)
                def _row(i):
                    for j in range(D // LANES):
                        sl = pl.ds(j * LANES, LANES)
                        obuf[b, i, sl] = gbuf[b, i, sl] * SCALE

                pltpu.async_copy(
                    obuf.at[b],
                    out_hbm.at[pl.ds(base + c * CHUNK, CHUNK)],
                    ssem[b],
                )

                @pl.when(c0 + nbuf < nchunks)
                def _():
                    gather(c + nbuf, b)

        for b in range(nbuf):
            c = nchunks - nbuf + b
            pltpu.make_async_copy(
                obuf.at[b],
                out_hbm.at[pl.ds(base + c * CHUNK, CHUNK)],
                ssem[b],
            ).wait()

    return k


def kernel(x, table):
    info = plsc.get_sparse_core_info()
    nc, ns = info.num_cores, info.num_subcores
    nw = nc * ns
    orig_shape = x.shape
    b = x.size
    xf = x.reshape(-1).astype(jnp.int32)
    block = nw * CHUNK
    pad = (-b) % block
    if pad:
        xf = jnp.concatenate([xf, jnp.zeros((pad,), jnp.int32)])
    tp = jnp.pad(table, ((0, 0), (0, 128 - D_MODEL)))
    out = _build(nw, nc, b + pad, tp.shape[0])(xf, tp)
    if pad:
        out = out[:b]
    return out.reshape(*orig_shape, D_MODEL)
